# baseline (device time: 31595 ns/iter reference)
import os

import jax
import jax.numpy as jnp
from jax import lax
from jax.experimental import pallas as pl
from jax.experimental.pallas import tpu as pltpu

_DEBUG_NO_RDMA = os.environ.get("SCBAND_DEBUG_NO_RDMA") == "1"
_DEBUG_SKIP = os.environ.get("SCBAND_DEBUG_SKIP", "")

N_DEV = 4
B = 2
S = 512
W = 128
HQ = 8
DH = 64
HD = HQ * DH
E = 768
S_GLOBAL = N_DEV * S


def kernel(x, Wq, K_ext, V_ext, Wo):
    K_t = K_ext.transpose(0, 2, 3, 1)
    V_t = V_ext.transpose(0, 2, 3, 1)

    def body(x_ref, wq_ref, kt_ref, vt_ref, wo_ref, out_hbm,
             own_k, own_v, lk, lv, rk, rv, skl, skr, svl, svr,
             out_vmem, send_sems, recv_sems, out_sems):
        my = lax.axis_index("i")
        left = lax.rem(my + N_DEV - 1, N_DEV)
        right = lax.rem(my + 1, N_DEV)

        own_k[...] = kt_ref[...].astype(jnp.bfloat16)
        own_v[...] = vt_ref[...].astype(jnp.bfloat16)
        skl[...] = own_k[:, :, :, 0:W]
        skr[...] = own_k[:, :, :, S - W:S]
        svl[...] = own_v[:, :, :, 0:W]
        svr[...] = own_v[:, :, :, S - W:S]

        barrier = pltpu.get_barrier_semaphore()
        for nbr in (left, right):
            pl.semaphore_signal(
                barrier, inc=1,
                device_id=(nbr,), device_id_type=pl.DeviceIdType.MESH,
            )
        pl.semaphore_wait(barrier, 2)

        flows = [] if _DEBUG_NO_RDMA else [
            (skr, lk, right),
            (svr, lv, right),
            (skl, rk, left),
            (svl, rv, left),
        ]
        rdmas = []
        for idx, (src, dst, tgt) in enumerate(flows):
            rdma = pltpu.make_async_remote_copy(
                src_ref=src, dst_ref=dst,
                send_sem=send_sems.at[idx], recv_sem=recv_sems.at[idx],
                device_id=(tgt,), device_id_type=pl.DeviceIdType.MESH,
            )
            rdma.start()
            rdmas.append(rdma)

        wq_b = wq_ref[...].astype(jnp.bfloat16)
        wo_b = wo_ref[...].astype(jnp.bfloat16)
        q_heads = []
        for b in range(B):
            xb = x_ref[b].astype(jnp.bfloat16)
            q_heads.append([
                lax.dot_general(
                    xb, wq_b[:, h * DH:(h + 1) * DH],
                    (((1,), (0,)), ((), ())),
                    preferred_element_type=jnp.float32,
                ).astype(jnp.bfloat16)
                for h in range(HQ)
            ])

        BLOCKS = {
            "left": (0, W, 3 * W),
            "mid": (W, S - 2 * W, S),
            "right": (S - W, W, 3 * W),
        }
        biases = {}
        for name, (r0, rw, kw) in BLOCKS.items():
            r_io = lax.broadcasted_iota(jnp.int32, (rw, kw), 0)
            c_io = lax.broadcasted_iota(jnp.int32, (rw, kw), 1)
            band = (c_io - r_io >= 0) & (c_io - r_io <= 2 * W)
            kpos = my * S - W + r0 + c_io
            valid = band & (kpos >= 0) & (kpos < S_GLOBAL)
            biases[name] = jnp.where(valid, 0.0, -1e9).astype(jnp.float32)

        def block(b, name):
            r0, rw, kw = BLOCKS[name]
            if _DEBUG_SKIP == "attn":
                ctx = jnp.concatenate(
                    [q_heads[b][h][r0:r0 + rw, :] for h in range(HQ)], axis=1)
            else:
                ctx_heads = []
                for h in range(HQ):
                    if name == "left":
                        kp = jnp.concatenate(
                            [lk[b, h], own_k[b, h, :, 0:2 * W]], axis=1)
                        vp = jnp.concatenate(
                            [lv[b, h], own_v[b, h, :, 0:2 * W]], axis=1)
                    elif name == "mid":
                        kp = own_k[b, h]
                        vp = own_v[b, h]
                    else:
                        kp = jnp.concatenate(
                            [own_k[b, h, :, S - 2 * W:S], rk[b, h]], axis=1)
                        vp = jnp.concatenate(
                            [own_v[b, h, :, S - 2 * W:S], rv[b, h]], axis=1)
                    qh = q_heads[b][h][r0:r0 + rw, :]
                    s = lax.dot_general(
                        qh, kp, (((1,), (0,)), ((), ())),
                        preferred_element_type=jnp.float32,
                    )
                    if _DEBUG_SKIP == "softmax":
                        e = s.astype(jnp.bfloat16)
                        inv = 1.0
                    else:
                        e = jnp.exp(s * 0.125 + biases[name])
                        inv = 1.0 / lax.dot_general(
                            e.astype(jnp.bfloat16),
                            jnp.ones((kw, 1), jnp.bfloat16),
                            (((1,), (0,)), ((), ())),
                            preferred_element_type=jnp.float32,
                        )
                        e = e.astype(jnp.bfloat16)
                    ctx_u = lax.dot_general(
                        e, vp, (((1,), (1,)), ((), ())),
                        preferred_element_type=jnp.float32,
                    )
                    ctx_heads.append((ctx_u * inv).astype(jnp.bfloat16))
                ctx = jnp.concatenate(ctx_heads, axis=1)
            out_vmem[b, pl.ds(r0, rw), :] = lax.dot_general(
                ctx, wo_b, (((1,), (0,)), ((), ())),
                preferred_element_type=jnp.float32,
            ).astype(jnp.bfloat16)

        out_dmas = []

        def flush(b, name):
            r0, rw, _ = BLOCKS[name]
            cp = pltpu.make_async_copy(
                out_vmem.at[b, pl.ds(r0, rw), :],
                out_hbm.at[b, pl.ds(r0, rw), :],
                out_sems.at[len(out_dmas)],
            )
            cp.start()
            out_dmas.append(cp)

        for b in range(B):
            block(b, "mid")
            flush(b, "mid")
        if not _DEBUG_NO_RDMA:
            rdmas[0].wait_recv()
            rdmas[1].wait_recv()
        for b in range(B):
            block(b, "left")
            flush(b, "left")
        if not _DEBUG_NO_RDMA:
            rdmas[2].wait_recv()
            rdmas[3].wait_recv()
        for b in range(B):
            block(b, "right")
            flush(b, "right")

        for cp in out_dmas:
            cp.wait()
        for r in rdmas:
            r.wait_send()

    return pl.pallas_call(
        body,
        out_shape=jax.ShapeDtypeStruct((B, S, E), jnp.bfloat16),
        in_specs=[pl.BlockSpec(memory_space=pltpu.VMEM)] * 5,
        out_specs=pl.BlockSpec(memory_space=pl.ANY),
        scratch_shapes=[
            pltpu.VMEM((B, HQ, DH, S), jnp.bfloat16),
            pltpu.VMEM((B, HQ, DH, S), jnp.bfloat16),
            pltpu.VMEM((B, HQ, DH, W), jnp.bfloat16),
            pltpu.VMEM((B, HQ, DH, W), jnp.bfloat16),
            pltpu.VMEM((B, HQ, DH, W), jnp.bfloat16),
            pltpu.VMEM((B, HQ, DH, W), jnp.bfloat16),
            pltpu.VMEM((B, HQ, DH, W), jnp.bfloat16),
            pltpu.VMEM((B, HQ, DH, W), jnp.bfloat16),
            pltpu.VMEM((B, HQ, DH, W), jnp.bfloat16),
            pltpu.VMEM((B, HQ, DH, W), jnp.bfloat16),
            pltpu.VMEM((B, S, E), jnp.bfloat16),
            pltpu.SemaphoreType.DMA((4,)),
            pltpu.SemaphoreType.DMA((4,)),
            pltpu.SemaphoreType.DMA((3 * B,)),
        ],
        compiler_params=pltpu.CompilerParams(collective_id=0),
    )(x, Wq, K_t, V_t, Wo)


# device time: 29559 ns/iter; 1.0689x vs baseline; 1.0689x over previous
import os

import jax
import jax.numpy as jnp
from jax import lax
from jax.experimental import pallas as pl
from jax.experimental.pallas import tpu as pltpu

_DEBUG_NO_RDMA = os.environ.get("SCBAND_DEBUG_NO_RDMA") == "1"
_DEBUG_SKIP = os.environ.get("SCBAND_DEBUG_SKIP", "")

N_DEV = 4
B = 2
S = 512
W = 128
HQ = 8
DH = 64
HD = HQ * DH
E = 768
S_EXT = S + 2 * W
S_GLOBAL = N_DEV * S


def kernel(x, Wq, K_ext, V_ext, Wo):
    K_t = K_ext.transpose(0, 2, 3, 1)
    V_t = V_ext.transpose(0, 2, 3, 1)

    def body(x_ref, wq_ref, kt_ref, vt_ref, wo_ref, out_hbm,
             keff, veff, lk, lv, rk, rv, skl, skr, svl, svr,
             out_vmem, send_sems, recv_sems, out_sems):
        my = lax.axis_index("i")
        left = lax.rem(my + N_DEV - 1, N_DEV)
        right = lax.rem(my + 1, N_DEV)

        keff[:, :, :, W:W + S] = kt_ref[...].astype(jnp.bfloat16)
        veff[:, :, :, W:W + S] = vt_ref[...].astype(jnp.bfloat16)
        skl[...] = keff[:, :, :, W:2 * W]
        skr[...] = keff[:, :, :, S:S + W]
        svl[...] = veff[:, :, :, W:2 * W]
        svr[...] = veff[:, :, :, S:S + W]

        barrier = pltpu.get_barrier_semaphore()
        for nbr in (left, right):
            pl.semaphore_signal(
                barrier, inc=1,
                device_id=(nbr,), device_id_type=pl.DeviceIdType.MESH,
            )
        pl.semaphore_wait(barrier, 2)

        flows = [] if _DEBUG_NO_RDMA else [
            (skr, lk, right),
            (svr, lv, right),
            (skl, rk, left),
            (svl, rv, left),
        ]
        rdmas = []
        for idx, (src, dst, tgt) in enumerate(flows):
            rdma = pltpu.make_async_remote_copy(
                src_ref=src, dst_ref=dst,
                send_sem=send_sems.at[idx], recv_sem=recv_sems.at[idx],
                device_id=(tgt,), device_id_type=pl.DeviceIdType.MESH,
            )
            rdma.start()
            rdmas.append(rdma)

        wq_b = wq_ref[...].astype(jnp.bfloat16)
        wo_b = wo_ref[...].astype(jnp.bfloat16)
        q_all = []
        for b in range(B):
            xb = x_ref[b].astype(jnp.bfloat16)
            qb = lax.dot_general(
                xb, wq_b, (((1,), (0,)), ((), ())),
                preferred_element_type=jnp.float32,
            )
            q_all.append(qb.astype(jnp.bfloat16))

        BLOCKS = {
            "left": (0, W, 3 * W),
            "mid": (W, S - 2 * W, S),
            "right": (S - W, W, 3 * W),
        }
        biases = {}
        for name, (r0, rw, kw) in BLOCKS.items():
            r_io = lax.broadcasted_iota(jnp.int32, (rw, kw), 0)
            c_io = lax.broadcasted_iota(jnp.int32, (rw, kw), 1)
            band = (c_io - r_io >= 0) & (c_io - r_io <= 2 * W)
            kpos = my * S - W + r0 + c_io
            valid = band & (kpos >= 0) & (kpos < S_GLOBAL)
            biases[name] = jnp.where(valid, 0.0, -1e9).astype(jnp.float32)

        def block(b, name):
            r0, rw, kw = BLOCKS[name]
            if _DEBUG_SKIP == "attn":
                ctx = q_all[b][r0:r0 + rw, :]
            else:
                ctx_heads = []
                for h in range(HQ):
                    kp = keff[b, h, :, pl.ds(r0, kw)]
                    vp = veff[b, h, :, pl.ds(r0, kw)]
                    qh = q_all[b][r0:r0 + rw, h * DH:(h + 1) * DH]
                    s = lax.dot_general(
                        qh, kp, (((1,), (0,)), ((), ())),
                        preferred_element_type=jnp.float32,
                    )
                    if _DEBUG_SKIP == "softmax":
                        e = s.astype(jnp.bfloat16)
                        inv = 1.0
                    else:
                        e = jnp.exp(s * 0.125 + biases[name])
                        inv = 1.0 / lax.dot_general(
                            e.astype(jnp.bfloat16),
                            jnp.ones((kw, 1), jnp.bfloat16),
                            (((1,), (0,)), ((), ())),
                            preferred_element_type=jnp.float32,
                        )
                        e = e.astype(jnp.bfloat16)
                    ctx_u = lax.dot_general(
                        e, vp, (((1,), (1,)), ((), ())),
                        preferred_element_type=jnp.float32,
                    )
                    ctx_heads.append((ctx_u * inv).astype(jnp.bfloat16))
                ctx = jnp.concatenate(ctx_heads, axis=1)
            out_vmem[b, pl.ds(r0, rw), :] = lax.dot_general(
                ctx, wo_b, (((1,), (0,)), ((), ())),
                preferred_element_type=jnp.float32,
            ).astype(jnp.bfloat16)

        out_dmas = []

        def flush(b, name):
            r0, rw, _ = BLOCKS[name]
            cp = pltpu.make_async_copy(
                out_vmem.at[b, pl.ds(r0, rw), :],
                out_hbm.at[b, pl.ds(r0, rw), :],
                out_sems.at[len(out_dmas)],
            )
            cp.start()
            out_dmas.append(cp)

        for b in range(B):
            block(b, "mid")
            flush(b, "mid")
        if not _DEBUG_NO_RDMA:
            rdmas[0].wait_recv()
            rdmas[1].wait_recv()
        keff[:, :, :, 0:W] = lk[...]
        veff[:, :, :, 0:W] = lv[...]
        for b in range(B):
            block(b, "left")
            flush(b, "left")
        if not _DEBUG_NO_RDMA:
            rdmas[2].wait_recv()
            rdmas[3].wait_recv()
        keff[:, :, :, S + W:S_EXT] = rk[...]
        veff[:, :, :, S + W:S_EXT] = rv[...]
        for b in range(B):
            block(b, "right")
            flush(b, "right")

        for cp in out_dmas:
            cp.wait()
        for r in rdmas:
            r.wait_send()

    return pl.pallas_call(
        body,
        out_shape=jax.ShapeDtypeStruct((B, S, E), jnp.bfloat16),
        in_specs=[pl.BlockSpec(memory_space=pltpu.VMEM)] * 5,
        out_specs=pl.BlockSpec(memory_space=pl.ANY),
        scratch_shapes=[
            pltpu.VMEM((B, HQ, DH, S_EXT), jnp.bfloat16),
            pltpu.VMEM((B, HQ, DH, S_EXT), jnp.bfloat16),
            pltpu.VMEM((B, HQ, DH, W), jnp.bfloat16),
            pltpu.VMEM((B, HQ, DH, W), jnp.bfloat16),
            pltpu.VMEM((B, HQ, DH, W), jnp.bfloat16),
            pltpu.VMEM((B, HQ, DH, W), jnp.bfloat16),
            pltpu.VMEM((B, HQ, DH, W), jnp.bfloat16),
            pltpu.VMEM((B, HQ, DH, W), jnp.bfloat16),
            pltpu.VMEM((B, HQ, DH, W), jnp.bfloat16),
            pltpu.VMEM((B, HQ, DH, W), jnp.bfloat16),
            pltpu.VMEM((B, S, E), jnp.bfloat16),
            pltpu.SemaphoreType.DMA((4,)),
            pltpu.SemaphoreType.DMA((4,)),
            pltpu.SemaphoreType.DMA((3 * B,)),
        ],
        compiler_params=pltpu.CompilerParams(collective_id=0),
    )(x, Wq, K_t, V_t, Wo)


# device time: 26193 ns/iter; 1.2062x vs baseline; 1.1285x over previous
import os

import jax
import jax.numpy as jnp
from jax import lax
from jax.experimental import pallas as pl
from jax.experimental.pallas import tpu as pltpu

_DEBUG_NO_RDMA = os.environ.get("SCBAND_DEBUG_NO_RDMA") == "1"
_DEBUG_SKIP = os.environ.get("SCBAND_DEBUG_SKIP", "")

N_DEV = 4
B = 2
S = 512
W = 128
HQ = 8
DH = 64
HD = HQ * DH
E = 768
S_EXT = S + 2 * W
S_GLOBAL = N_DEV * S
QBLK = 256
KSLAB = QBLK + 2 * W
NQB = S // QBLK


def kernel(x, Wq, K_ext, V_ext, Wo):
    K2 = K_ext.reshape(B, S, HD)
    V2 = V_ext.reshape(B, S, HD)

    def body(x_ref, wq_ref, k_ref, v_ref, wo_ref, out_hbm,
             keff, veff, out_vmem, send_sems, recv_sems, out_sems):
        my = lax.axis_index("i")
        left = lax.rem(my + N_DEV - 1, N_DEV)
        right = lax.rem(my + 1, N_DEV)

        keff[:, W:W + S, :] = k_ref[...].astype(jnp.bfloat16)
        veff[:, W:W + S, :] = v_ref[...].astype(jnp.bfloat16)

        barrier = pltpu.get_barrier_semaphore()
        for nbr in (left, right):
            pl.semaphore_signal(
                barrier, inc=1,
                device_id=(nbr,), device_id_type=pl.DeviceIdType.MESH,
            )
        pl.semaphore_wait(barrier, 2)

        flows = [] if _DEBUG_NO_RDMA else [
            (buf, b, src_lo, dst_lo, tgt)
            for b in range(B)
            for (buf, src_lo, dst_lo, tgt) in [
                (keff, S, 0, right),
                (veff, S, 0, right),
                (keff, W, S + W, left),
                (veff, W, S + W, left),
            ]
        ]
        rdmas = []
        for idx, (buf, b, src_lo, dst_lo, tgt) in enumerate(flows):
            rdma = pltpu.make_async_remote_copy(
                src_ref=buf.at[b, pl.ds(src_lo, W), :],
                dst_ref=buf.at[b, pl.ds(dst_lo, W), :],
                send_sem=send_sems.at[idx],
                recv_sem=recv_sems.at[idx],
                device_id=(tgt,),
                device_id_type=pl.DeviceIdType.MESH,
            )
            rdma.start()
            rdmas.append(rdma)
        left_halo = None if _DEBUG_NO_RDMA else [
            (rdmas[b * 4 + 0], rdmas[b * 4 + 1]) for b in range(B)]
        right_halo = None if _DEBUG_NO_RDMA else [
            (rdmas[b * 4 + 2], rdmas[b * 4 + 3]) for b in range(B)]

        wq_b = wq_ref[...].astype(jnp.bfloat16)
        wo_b = wo_ref[...].astype(jnp.bfloat16)
        q_all = []
        for b in range(B):
            xb = x_ref[b].astype(jnp.bfloat16)
            qb = lax.dot_general(
                xb, wq_b, (((1,), (0,)), ((), ())),
                preferred_element_type=jnp.float32,
            )
            q_all.append(qb.astype(jnp.bfloat16))

        BLOCKS = {
            "left": (0, W, 3 * W),
            "mid": (W, S - 2 * W, S),
            "right": (S - W, W, 3 * W),
        }
        biases = {}
        for name, (r0, rw, kw) in BLOCKS.items():
            r_io = lax.broadcasted_iota(jnp.int32, (rw, kw), 0)
            c_io = lax.broadcasted_iota(jnp.int32, (rw, kw), 1)
            band = (c_io - r_io >= 0) & (c_io - r_io <= 2 * W)
            kpos = my * S - W + r0 + c_io
            valid = band & (kpos >= 0) & (kpos < S_GLOBAL)
            biases[name] = jnp.where(valid, 0.0, -1e9).astype(jnp.float32)

        def block(b, name):
            r0, rw, kw = BLOCKS[name]
            kslab = keff[b, pl.ds(r0, kw), :]
            vslab = veff[b, pl.ds(r0, kw), :]
            if _DEBUG_SKIP == "attn":
                ctx = q_all[b][r0:r0 + rw, :]
                out_vmem[b, pl.ds(r0, rw), :] = lax.dot_general(
                    ctx, wo_b, (((1,), (0,)), ((), ())),
                    preferred_element_type=jnp.float32,
                ).astype(jnp.bfloat16)
                return
            ctx_heads = []
            for h in range(HQ):
                qh = q_all[b][r0:r0 + rw, h * DH:(h + 1) * DH]
                kh = kslab[:, h * DH:(h + 1) * DH]
                vh = vslab[:, h * DH:(h + 1) * DH]
                s = lax.dot_general(
                    qh, kh, (((1,), (1,)), ((), ())),
                    preferred_element_type=jnp.float32,
                )
                if _DEBUG_SKIP == "softmax":
                    e = s.astype(jnp.bfloat16)
                    inv = 1.0
                else:
                    e = jnp.exp(s * 0.125 + biases[name])
                    inv = 1.0 / lax.dot_general(
                        e.astype(jnp.bfloat16),
                        jnp.ones((kw, 1), jnp.bfloat16),
                        (((1,), (0,)), ((), ())),
                        preferred_element_type=jnp.float32,
                    )
                    e = e.astype(jnp.bfloat16)
                ctx_u = lax.dot_general(
                    e, vh, (((1,), (0,)), ((), ())),
                    preferred_element_type=jnp.float32,
                )
                ctx_heads.append((ctx_u * inv).astype(jnp.bfloat16))
            ctx = jnp.concatenate(ctx_heads, axis=1)
            out_vmem[b, pl.ds(r0, rw), :] = lax.dot_general(
                ctx, wo_b, (((1,), (0,)), ((), ())),
                preferred_element_type=jnp.float32,
            ).astype(jnp.bfloat16)

        out_dmas = []

        def flush(b, name):
            r0, rw, _ = BLOCKS[name]
            cp = pltpu.make_async_copy(
                out_vmem.at[b, pl.ds(r0, rw), :],
                out_hbm.at[b, pl.ds(r0, rw), :],
                out_sems.at[len(out_dmas)],
            )
            cp.start()
            out_dmas.append(cp)

        for b in range(B):
            block(b, "mid")
            flush(b, "mid")
        for b in range(B):
            if left_halo is not None:
                for r in left_halo[b]:
                    r.wait_recv()
            block(b, "left")
            flush(b, "left")
        for b in range(B):
            if right_halo is not None:
                for r in right_halo[b]:
                    r.wait_recv()
            block(b, "right")
            flush(b, "right")

        for cp in out_dmas:
            cp.wait()
        for r in rdmas:
            r.wait_send()

    return pl.pallas_call(
        body,
        out_shape=jax.ShapeDtypeStruct((B, S, E), jnp.bfloat16),
        in_specs=[pl.BlockSpec(memory_space=pltpu.VMEM)] * 5,
        out_specs=pl.BlockSpec(memory_space=pl.ANY),
        scratch_shapes=[
            pltpu.VMEM((B, S_EXT, HD), jnp.bfloat16),
            pltpu.VMEM((B, S_EXT, HD), jnp.bfloat16),
            pltpu.VMEM((B, S, E), jnp.bfloat16),
            pltpu.SemaphoreType.DMA((4 * B,)),
            pltpu.SemaphoreType.DMA((4 * B,)),
            pltpu.SemaphoreType.DMA((3 * B,)),
        ],
        compiler_params=pltpu.CompilerParams(collective_id=0),
    )(x, Wq, K2, V2, Wo)


# device time: 24292 ns/iter; 1.3006x vs baseline; 1.0783x over previous
import os

import jax
import jax.numpy as jnp
from jax import lax
from jax.experimental import pallas as pl
from jax.experimental.pallas import tpu as pltpu

_DEBUG_NO_RDMA = os.environ.get("SCBAND_DEBUG_NO_RDMA") == "1"
_DEBUG_SKIP = os.environ.get("SCBAND_DEBUG_SKIP", "")

N_DEV = 4
B = 2
S = 512
W = 128
HQ = 8
DH = 64
HD = HQ * DH
E = 768
S_EXT = S + 2 * W
S_GLOBAL = N_DEV * S
QBLK = 256
KSLAB = QBLK + 2 * W
NQB = S // QBLK


def kernel(x, Wq, K_ext, V_ext, Wo):
    K2 = K_ext.astype(jnp.bfloat16).reshape(B, S, HD)
    V2 = V_ext.astype(jnp.bfloat16).reshape(B, S, HD)

    def body(x_ref, wq_ref, k_ref, v_ref, wo_ref, out_ref,
             keff, veff, send_sems, recv_sems):
        my = lax.axis_index("i")
        left = lax.rem(my + N_DEV - 1, N_DEV)
        right = lax.rem(my + 1, N_DEV)

        keff[:, W:W + S, :] = k_ref[...]
        veff[:, W:W + S, :] = v_ref[...]

        barrier = pltpu.get_barrier_semaphore()
        for nbr in (left, right):
            pl.semaphore_signal(
                barrier, inc=1,
                device_id=(nbr,), device_id_type=pl.DeviceIdType.MESH,
            )
        pl.semaphore_wait(barrier, 2)

        flows = [] if _DEBUG_NO_RDMA else [
            (buf, b, src_lo, dst_lo, tgt)
            for b in range(B)
            for (buf, src_lo, dst_lo, tgt) in [
                (keff, S, 0, right),
                (veff, S, 0, right),
                (keff, W, S + W, left),
                (veff, W, S + W, left),
            ]
        ]
        rdmas = []
        for idx, (buf, b, src_lo, dst_lo, tgt) in enumerate(flows):
            rdma = pltpu.make_async_remote_copy(
                src_ref=buf.at[b, pl.ds(src_lo, W), :],
                dst_ref=buf.at[b, pl.ds(dst_lo, W), :],
                send_sem=send_sems.at[idx],
                recv_sem=recv_sems.at[idx],
                device_id=(tgt,),
                device_id_type=pl.DeviceIdType.MESH,
            )
            rdma.start()
            rdmas.append(rdma)
        left_halo = None if _DEBUG_NO_RDMA else [
            (rdmas[b * 4 + 0], rdmas[b * 4 + 1]) for b in range(B)]
        right_halo = None if _DEBUG_NO_RDMA else [
            (rdmas[b * 4 + 2], rdmas[b * 4 + 3]) for b in range(B)]

        wq_b = wq_ref[...].astype(jnp.bfloat16)
        wo_b = wo_ref[...].astype(jnp.bfloat16)
        q_all = []
        for b in range(B):
            xb = x_ref[b].astype(jnp.bfloat16)
            qb = lax.dot_general(
                xb, wq_b, (((1,), (0,)), ((), ())),
                preferred_element_type=jnp.float32,
            )
            q_all.append(qb.astype(jnp.bfloat16))

        BLOCKS = {
            "left": (0, W, 3 * W),
            "mid": (W, S - 2 * W, S),
            "right": (S - W, W, 3 * W),
        }
        biases = {}
        for name, (r0, rw, kw) in BLOCKS.items():
            r_io = lax.broadcasted_iota(jnp.int32, (rw, kw), 0)
            c_io = lax.broadcasted_iota(jnp.int32, (rw, kw), 1)
            band = (c_io - r_io >= 0) & (c_io - r_io <= 2 * W)
            kpos = my * S - W + r0 + c_io
            valid = band & (kpos >= 0) & (kpos < S_GLOBAL)
            biases[name] = jnp.where(valid, 0.0, -1e9).astype(jnp.float32)

        def block(b, name):
            r0, rw, kw = BLOCKS[name]
            kslab = keff[b, pl.ds(r0, kw), :]
            vslab = veff[b, pl.ds(r0, kw), :]
            if _DEBUG_SKIP == "attn":
                ctx = q_all[b][r0:r0 + rw, :]
                out_ref[b, pl.ds(r0, rw), :] = lax.dot_general(
                    ctx, wo_b, (((1,), (0,)), ((), ())),
                    preferred_element_type=jnp.float32,
                ).astype(jnp.bfloat16)
                return
            ctx_heads = []
            for h in range(HQ):
                qh = q_all[b][r0:r0 + rw, h * DH:(h + 1) * DH]
                kh = kslab[:, h * DH:(h + 1) * DH]
                vh = vslab[:, h * DH:(h + 1) * DH]
                s = lax.dot_general(
                    qh, kh, (((1,), (1,)), ((), ())),
                    preferred_element_type=jnp.float32,
                )
                if _DEBUG_SKIP == "softmax":
                    e = s.astype(jnp.bfloat16)
                    inv = 1.0
                else:
                    e = jnp.exp(s * 0.125 + biases[name])
                    inv = 1.0 / lax.dot_general(
                        e.astype(jnp.bfloat16),
                        jnp.ones((kw, 1), jnp.bfloat16),
                        (((1,), (0,)), ((), ())),
                        preferred_element_type=jnp.float32,
                    )
                    e = e.astype(jnp.bfloat16)
                ctx_u = lax.dot_general(
                    e, vh, (((1,), (0,)), ((), ())),
                    preferred_element_type=jnp.float32,
                )
                ctx_heads.append((ctx_u * inv).astype(jnp.bfloat16))
            ctx = jnp.concatenate(ctx_heads, axis=1)
            out_ref[b, pl.ds(r0, rw), :] = lax.dot_general(
                ctx, wo_b, (((1,), (0,)), ((), ())),
                preferred_element_type=jnp.float32,
            ).astype(jnp.bfloat16)

        for b in range(B):
            block(b, "mid")
        for b in range(B):
            if left_halo is not None:
                for r in left_halo[b]:
                    r.wait_recv()
            block(b, "left")
        for b in range(B):
            if right_halo is not None:
                for r in right_halo[b]:
                    r.wait_recv()
            block(b, "right")

        for r in rdmas:
            r.wait_send()

    return pl.pallas_call(
        body,
        out_shape=jax.ShapeDtypeStruct((B, S, E), jnp.bfloat16),
        in_specs=[pl.BlockSpec(memory_space=pltpu.VMEM)] * 5,
        out_specs=pl.BlockSpec(memory_space=pltpu.VMEM),
        scratch_shapes=[
            pltpu.VMEM((B, S_EXT, HD), jnp.bfloat16),
            pltpu.VMEM((B, S_EXT, HD), jnp.bfloat16),
            pltpu.SemaphoreType.DMA((4 * B,)),
            pltpu.SemaphoreType.DMA((4 * B,)),
        ],
        compiler_params=pltpu.CompilerParams(collective_id=0),
    )(x, Wq, K2, V2, Wo)
